# baseline (speedup 1.0000x reference)
import functools

import jax
import jax.numpy as jnp
from jax import lax
from jax.experimental import pallas as pl
from jax.experimental.pallas import tpu as pltpu
from jax.experimental.pallas import tpu_sc as plsc

BATCH = 16384
DIM = 128
LANES = 16
CHUNK = 128
GROUP = 16


@functools.cache
def _build():
    info = plsc.get_sparse_core_info()
    nc, ns = info.num_cores, info.num_subcores
    nw = nc * ns
    b_per_w = BATCH // nw
    n_chunks = b_per_w // CHUNK
    n_groups = CHUNK // GROUP
    n_seg = DIM // LANES

    mesh = plsc.VectorSubcoreMesh(core_axis_name="c", subcore_axis_name="s")

    @functools.partial(
        pl.kernel,
        mesh=mesh,
        out_type=jax.ShapeDtypeStruct((BATCH,), jnp.float32),
        scratch_types=[
            pltpu.VMEM((b_per_w,), jnp.int32),
            pltpu.VMEM((CHUNK, DIM), jnp.float32),
            pltpu.VMEM((CHUNK, DIM), jnp.float32),
            pltpu.VMEM((CHUNK, DIM), jnp.float32),
            pltpu.VMEM((CHUNK, DIM), jnp.float32),
            pltpu.VMEM((b_per_w,), jnp.float32),
            pltpu.SemaphoreType.DMA,
            pltpu.SemaphoreType.DMA,
            pltpu.SemaphoreType.DMA,
            pltpu.SemaphoreType.DMA,
            pltpu.SemaphoreType.DMA,
        ],
    )
    def sc_kernel(x_hbm, label_hbm, table_hbm, out_hbm,
                  idx_v, x_v0, x_v1, e_v0, e_v1, o_v,
                  sem_l, sem_x0, sem_x1, sem_g0, sem_g1):
        wid = lax.axis_index("s") * nc + lax.axis_index("c")
        base = wid * b_per_w
        lane = lax.iota(jnp.int32, LANES)
        perms = [lane ^ jnp.int32(k) for k in (1, 2, 4, 8)]
        masks = [(lane & jnp.int32(k)) == 0 for k in (1, 2, 4, 8)]

        gdn = lax.GatherDimensionNumbers(
            offset_dims=(), collapsed_slice_dims=(0,), start_index_map=(0,))

        def lane_perm(v, perm):
            return lax.gather(
                v, perm[:, None], dimension_numbers=gdn, slice_sizes=(1,),
                mode=lax.GatherScatterMode.PROMISE_IN_BOUNDS)

        x_bufs, e_bufs = [x_v0, x_v1], [e_v0, e_v1]
        x_sems, g_sems = [sem_x0, sem_x1], [sem_g0, sem_g1]

        def start_copies(c):
            b = c % 2
            xcp = pltpu.async_copy(
                x_hbm.at[pl.ds(base + c * CHUNK, CHUNK)], x_bufs[b], x_sems[b])
            gcp = pltpu.async_copy(
                table_hbm.at[idx_v.at[pl.ds(c * CHUNK, CHUNK)]],
                e_bufs[b], g_sems[b])
            return xcp, gcp

        lcp = pltpu.async_copy(label_hbm.at[pl.ds(base, b_per_w)], idx_v, sem_l)
        lcp.wait()
        cps = {0: start_copies(0)}

        for c in range(n_chunks):
            if c + 1 < n_chunks:
                cps[c + 1] = start_copies(c + 1)
            xcp, gcp = cps.pop(c)
            gcp.wait()
            xcp.wait()
            x_v, e_v = x_bufs[c % 2], e_bufs[c % 2]

            def group_body(g, carry2, c=c, x_v=x_v, e_v=e_v):
                vecs = []
                ilv = 2
                for jj in range(0, GROUP, ilv):
                    rows = [g * GROUP + jj + k for k in range(ilv)]
                    qs = [[] for _ in range(ilv)]
                    for t in range(n_seg):
                        xs = [x_v[rows[k], pl.ds(t * LANES, LANES)]
                              for k in range(ilv)]
                        es = [e_v[rows[k], pl.ds(t * LANES, LANES)]
                              for k in range(ilv)]
                        for k in range(ilv):
                            qs[k].append(xs[k] * es[k])
                    for q in qs:
                        while len(q) > 1:
                            q = [a + b for a, b in zip(q[0::2], q[1::2])]
                        vecs.append(q[0])
                for perm, m in zip(perms, masks):
                    vecs = [jnp.where(m, u, v)
                            + jnp.where(m, lane_perm(u, perm),
                                        lane_perm(v, perm))
                            for u, v in zip(vecs[0::2], vecs[1::2])]
                o_v[pl.ds(c * CHUNK + g * GROUP, GROUP)] = vecs[0]
                return carry2

            lax.fori_loop(0, n_groups, group_body, 0)

        pltpu.sync_copy(o_v, out_hbm.at[pl.ds(base, b_per_w)])

    return sc_kernel


def kernel(x, label, labelembed_weight):
    return _build()(x, label, labelembed_weight)


# speedup vs baseline: 1.0039x; 1.0039x over previous
import functools

import jax
import jax.numpy as jnp
from jax import lax
from jax.experimental import pallas as pl
from jax.experimental.pallas import tpu as pltpu
from jax.experimental.pallas import tpu_sc as plsc

BATCH = 16384
DIM = 128
LANES = 16
CHUNK = 128
GROUP = 16


@functools.cache
def _build():
    info = plsc.get_sparse_core_info()
    nc, ns = info.num_cores, info.num_subcores
    nw = nc * ns
    b_per_w = BATCH // nw
    n_chunks = b_per_w // CHUNK
    n_groups = CHUNK // GROUP
    n_seg = DIM // LANES

    mesh = plsc.VectorSubcoreMesh(core_axis_name="c", subcore_axis_name="s")

    @functools.partial(
        pl.kernel,
        mesh=mesh,
        out_type=jax.ShapeDtypeStruct((BATCH,), jnp.float32),
        scratch_types=[
            pltpu.VMEM((b_per_w,), jnp.int32),
            pltpu.VMEM((CHUNK, DIM), jnp.float32),
            pltpu.VMEM((CHUNK, DIM), jnp.float32),
            pltpu.VMEM((CHUNK, DIM), jnp.float32),
            pltpu.VMEM((CHUNK, DIM), jnp.float32),
            pltpu.VMEM((b_per_w,), jnp.float32),
            pltpu.SemaphoreType.DMA,
            pltpu.SemaphoreType.DMA,
            pltpu.SemaphoreType.DMA,
            pltpu.SemaphoreType.DMA,
            pltpu.SemaphoreType.DMA,
        ],
    )
    def sc_kernel(x_hbm, label_hbm, table_hbm, out_hbm,
                  idx_v, x_v0, x_v1, e_v0, e_v1, o_v,
                  sem_l, sem_x0, sem_x1, sem_g0, sem_g1):
        wid = lax.axis_index("s") * nc + lax.axis_index("c")
        base = wid * b_per_w
        lane = lax.iota(jnp.int32, LANES)
        perms = [lane ^ jnp.int32(k) for k in (1, 2, 4, 8)]
        masks = [(lane & jnp.int32(k)) == 0 for k in (1, 2, 4, 8)]

        gdn = lax.GatherDimensionNumbers(
            offset_dims=(), collapsed_slice_dims=(0,), start_index_map=(0,))

        def lane_perm(v, perm):
            return lax.gather(
                v, perm[:, None], dimension_numbers=gdn, slice_sizes=(1,),
                mode=lax.GatherScatterMode.PROMISE_IN_BOUNDS)

        x_bufs, e_bufs = [x_v0, x_v1], [e_v0, e_v1]
        x_sems, g_sems = [sem_x0, sem_x1], [sem_g0, sem_g1]

        def start_copies(c):
            b = c % 2
            xcp = pltpu.async_copy(
                x_hbm.at[pl.ds(base + c * CHUNK, CHUNK)], x_bufs[b], x_sems[b])
            gcp = pltpu.async_copy(
                table_hbm.at[idx_v.at[pl.ds(c * CHUNK, CHUNK)]],
                e_bufs[b], g_sems[b])
            return xcp, gcp

        lcp = pltpu.async_copy(label_hbm.at[pl.ds(base, b_per_w)], idx_v, sem_l)
        lcp.wait()
        cps = {0: start_copies(0)}

        for c in range(n_chunks):
            if c + 1 < n_chunks:
                cps[c + 1] = start_copies(c + 1)
            xcp, gcp = cps.pop(c)
            gcp.wait()
            xcp.wait()
            x_v, e_v = x_bufs[c % 2], e_bufs[c % 2]

            def group_body(g, carry2, c=c, x_v=x_v, e_v=e_v):
                vecs = []
                ilv = 2
                for jj in range(0, GROUP, ilv):
                    rows = [g * GROUP + jj + k for k in range(ilv)]
                    qs = [[] for _ in range(ilv)]
                    for t in range(n_seg):
                        for k in range(ilv):
                            qs[k].append(
                                x_v[rows[k], pl.ds(t * LANES, LANES)]
                                * e_v[rows[k], pl.ds(t * LANES, LANES)])
                    for q in qs:
                        while len(q) > 1:
                            q = [a + b for a, b in zip(q[0::2], q[1::2])]
                        vecs.append(q[0])
                for perm, m in zip(perms, masks):
                    vecs = [jnp.where(m, u, v)
                            + jnp.where(m, lane_perm(u, perm),
                                        lane_perm(v, perm))
                            for u, v in zip(vecs[0::2], vecs[1::2])]
                o_v[pl.ds(c * CHUNK + g * GROUP, GROUP)] = vecs[0]
                return carry2

            lax.fori_loop(0, n_groups, group_body, 0)

        pltpu.sync_copy(o_v, out_hbm.at[pl.ds(base, b_per_w)])

    return sc_kernel


def kernel(x, label, labelembed_weight):
    return _build()(x, label, labelembed_weight)


# speedup vs baseline: 1.0064x; 1.0025x over previous
import functools

import jax
import jax.numpy as jnp
from jax import lax
from jax.experimental import pallas as pl
from jax.experimental.pallas import tpu as pltpu
from jax.experimental.pallas import tpu_sc as plsc

BATCH = 16384
DIM = 128
LANES = 16
CHUNK = 128
GROUP = 16


@functools.cache
def _build():
    info = plsc.get_sparse_core_info()
    nc, ns = info.num_cores, info.num_subcores
    nw = nc * ns
    b_per_w = BATCH // nw
    n_chunks = b_per_w // CHUNK
    n_groups = CHUNK // GROUP
    n_seg = DIM // LANES

    mesh = plsc.VectorSubcoreMesh(core_axis_name="c", subcore_axis_name="s")

    @functools.partial(
        pl.kernel,
        mesh=mesh,
        out_type=jax.ShapeDtypeStruct((BATCH,), jnp.float32),
        scratch_types=[
            pltpu.VMEM((b_per_w,), jnp.int32),
            pltpu.VMEM((CHUNK, DIM), jnp.float32),
            pltpu.VMEM((CHUNK, DIM), jnp.float32),
            pltpu.VMEM((CHUNK, DIM), jnp.float32),
            pltpu.VMEM((CHUNK, DIM), jnp.float32),
            pltpu.VMEM((b_per_w,), jnp.float32),
            pltpu.SemaphoreType.DMA,
            pltpu.SemaphoreType.DMA,
            pltpu.SemaphoreType.DMA,
            pltpu.SemaphoreType.DMA,
            pltpu.SemaphoreType.DMA,
            pltpu.SemaphoreType.DMA,
        ],
    )
    def sc_kernel(x_hbm, label_hbm, table_hbm, out_hbm,
                  idx_v, x_v0, x_v1, e_v0, e_v1, o_v,
                  sem_l, sem_x0, sem_x1, sem_g0, sem_g1, sem_o):
        wid = lax.axis_index("s") * nc + lax.axis_index("c")
        base = wid * b_per_w
        lane = lax.iota(jnp.int32, LANES)
        perms = [lane ^ jnp.int32(k) for k in (1, 2, 4, 8)]
        masks = [(lane & jnp.int32(k)) == 0 for k in (1, 2, 4, 8)]

        gdn = lax.GatherDimensionNumbers(
            offset_dims=(), collapsed_slice_dims=(0,), start_index_map=(0,))

        def lane_perm(v, perm):
            return lax.gather(
                v, perm[:, None], dimension_numbers=gdn, slice_sizes=(1,),
                mode=lax.GatherScatterMode.PROMISE_IN_BOUNDS)

        x_bufs, e_bufs = [x_v0, x_v1], [e_v0, e_v1]
        x_sems, g_sems = [sem_x0, sem_x1], [sem_g0, sem_g1]

        def start_copies(c):
            b = c % 2
            xcp = pltpu.async_copy(
                x_hbm.at[pl.ds(base + c * CHUNK, CHUNK)], x_bufs[b], x_sems[b])
            gcp = pltpu.async_copy(
                table_hbm.at[idx_v.at[pl.ds(c * CHUNK, CHUNK)]],
                e_bufs[b], g_sems[b])
            return xcp, gcp

        lcp = pltpu.async_copy(label_hbm.at[pl.ds(base, b_per_w)], idx_v, sem_l)
        lcp.wait()
        cps = {0: start_copies(0)}
        ocps = []

        for c in range(n_chunks):
            if c + 1 < n_chunks:
                cps[c + 1] = start_copies(c + 1)
            xcp, gcp = cps.pop(c)
            gcp.wait()
            xcp.wait()
            x_v, e_v = x_bufs[c % 2], e_bufs[c % 2]

            def group_body(g, carry2, c=c, x_v=x_v, e_v=e_v):
                vecs = []
                ilv = 2
                for jj in range(0, GROUP, ilv):
                    rows = [g * GROUP + jj + k for k in range(ilv)]
                    qs = [[] for _ in range(ilv)]
                    for t in range(n_seg):
                        for k in range(ilv):
                            qs[k].append(
                                x_v[rows[k], pl.ds(t * LANES, LANES)]
                                * e_v[rows[k], pl.ds(t * LANES, LANES)])
                    for q in qs:
                        while len(q) > 1:
                            q = [a + b for a, b in zip(q[0::2], q[1::2])]
                        vecs.append(q[0])
                for perm, m in zip(perms, masks):
                    vecs = [jnp.where(m, u, v)
                            + jnp.where(m, lane_perm(u, perm),
                                        lane_perm(v, perm))
                            for u, v in zip(vecs[0::2], vecs[1::2])]
                o_v[pl.ds(c * CHUNK + g * GROUP, GROUP)] = vecs[0]
                return carry2

            lax.fori_loop(0, n_groups, group_body, 0)
            ocps.append(pltpu.async_copy(
                o_v.at[pl.ds(c * CHUNK, CHUNK)],
                out_hbm.at[pl.ds(base + c * CHUNK, CHUNK)], sem_o))

        for ocp in ocps:
            ocp.wait()

    return sc_kernel


def kernel(x, label, labelembed_weight):
    return _build()(x, label, labelembed_weight)
